# baseline (speedup 1.0000x reference)
import functools

import jax
import jax.numpy as jnp
from jax import lax
from jax.experimental import pallas as pl
from jax.experimental.pallas import tpu as pltpu
from jax.experimental.pallas import tpu_sc as plsc


def _make_gather_kernel(B, D, n_cores, n_subcores):
    nw = n_cores * n_subcores
    b_per_w = B // nw

    mesh = plsc.VectorSubcoreMesh(core_axis_name="c", subcore_axis_name="s")

    @functools.partial(
        pl.kernel,
        mesh=mesh,
        out_type=(
            jax.ShapeDtypeStruct((B, D), jnp.float32),
            jax.ShapeDtypeStruct((B, D), jnp.float32),
        ),
        scratch_types=[
            pltpu.VMEM((b_per_w,), jnp.int32),
            pltpu.VMEM((b_per_w // 2, D), jnp.float32),
            pltpu.VMEM((b_per_w // 2, D), jnp.float32),
            pltpu.SemaphoreType.DMA,
        ],
    )
    def gather_kernel(idx_hbm, mu_hbm, ls_hbm, mu_out, ls_out,
                      idx_v, mu_v, ls_v, sem):
        wid = lax.axis_index("s") * n_cores + lax.axis_index("c")
        base = pl.multiple_of(wid * b_per_w, b_per_w)
        pltpu.sync_copy(idx_hbm.at[pl.ds(base, b_per_w)], idx_v)

        half = b_per_w // 2
        grp = 16
        for h in range(2):
            def body(g, carry):
                v = idx_v[pl.ds(h * half + g * grp, grp)]
                for j in range(grp):
                    i = v[j]
                    pltpu.async_copy(mu_hbm.at[i], mu_v.at[g * grp + j], sem)
                    pltpu.async_copy(ls_hbm.at[i], ls_v.at[g * grp + j], sem)
                return carry

            lax.fori_loop(0, half // grp, body, 0)

            full = pl.ds(0, half)
            pltpu.make_async_copy(mu_hbm.at[full], mu_v, sem).wait()
            pltpu.make_async_copy(ls_hbm.at[full], ls_v, sem).wait()

            out_sl = pl.ds(base + h * half, half)
            pltpu.sync_copy(mu_v, mu_out.at[out_sl])
            pltpu.sync_copy(ls_v, ls_out.at[out_sl])

    return gather_kernel


def kernel(indices, mu, log_sigma):
    B = indices.shape[0]
    _, D = mu.shape
    info = plsc.get_sparse_core_info()
    gather = _make_gather_kernel(B, D, info.num_cores, info.num_subcores)
    return gather(indices.astype(jnp.int32), mu, log_sigma)
